# baseline (speedup 1.0000x reference)
import functools

import jax
import jax.numpy as jnp
from jax import lax
from jax.experimental import pallas as pl
from jax.experimental.pallas import tpu as pltpu
from jax.experimental.pallas import tpu_sc as plsc

D_MODEL = 64
_SCALE = 8.0
_CHUNK = 128
_LANES = 16
_NR = 8
_NO = 4
_K = 4
_PITCH = 129


_VOCAB = 1000000
_VB = 256


@functools.lru_cache(maxsize=None)
def _build_relayout():
    info = plsc.get_sparse_core_info()
    nw = info.num_cores * info.num_subcores
    n_blocks = _VOCAB // _VB
    tail_v = _VOCAB - n_blocks * _VB
    n_iter = -(-n_blocks // nw)
    n_step = -(-n_iter // 2)
    mesh = plsc.VectorSubcoreMesh(core_axis_name="c", subcore_axis_name="s")

    @functools.partial(
        pl.kernel,
        mesh=mesh,
        out_type=jax.ShapeDtypeStruct((_VOCAB // 2, 128), jnp.float32),
        scratch_types=[
            pltpu.VMEM((2, D_MODEL, _VB), jnp.float32),
            pltpu.VMEM((2, _VB // 2, 129), jnp.float32),
        ]
        + [pltpu.SemaphoreType.DMA] * 4,
        compiler_params=pltpu.CompilerParams(needs_layout_passes=False),
    )
    def relayout_kernel(tt_hbm, tail_hbm, out_hbm, in_v, buf_v, *sems):
        rsems, osems = sems[:2], sems[2:]
        wid = lax.axis_index("s") * info.num_cores + lax.axis_index("c")

        iota = lax.iota(jnp.int32, _LANES)
        rowsel = [
            lax.shift_right_logical(g * _LANES + iota, 1)
            for g in range(_VB // _LANES)
        ]
        colbase = [
            lax.bitwise_and(g * _LANES + iota, 1) * D_MODEL
            for g in range(_VB // _LANES)
        ]

        def gather(vb, j):
            pltpu.async_copy(
                tt_hbm.at[:, pl.ds(vb * _VB, _VB)], in_v.at[j], rsems[j]
            )

        def wait_in(j):
            pltpu.make_async_copy(
                tt_hbm.at[:, pl.ds(0, _VB)], in_v.at[j], rsems[j]
            ).wait()

        def stores(vb, j):
            pltpu.async_copy(
                buf_v.at[j, :, pl.ds(0, 128)],
                out_hbm.at[pl.ds(vb * (_VB // 2), _VB // 2), :],
                osems[j],
            )

        def wait_stores(j):
            pltpu.make_async_copy(
                buf_v.at[j, :, pl.ds(0, 128)],
                out_hbm.at[pl.ds(0, _VB // 2), :],
                osems[j],
            ).wait()

        def transpose_block(j):
            @plsc.parallel_loop(0, D_MODEL, unroll=8)
            def f_body(f):
                fsplat = jnp.full((_LANES,), f, jnp.int32)
                for g in range(_VB // _LANES):
                    vals = in_v[j, f, pl.ds(g * _LANES, _LANES)]
                    plsc.store_scatter(
                        buf_v.at[j], [rowsel[g], colbase[g] + fsplat], vals
                    )

        for j in range(2):
            gather(wid + nw * j, j)

        def step(s, carry):
            for j in range(2):
                i = s * 2 + j
                vb = wid + nw * i
                valid = vb < n_blocks

                @pl.when(valid)
                def _():
                    wait_in(j)

                @pl.when(jnp.logical_and(valid, i >= 2))
                def _():
                    wait_stores(j)

                @pl.when(valid)
                def _():
                    transpose_block(j)
                    stores(vb, j)

                vb2 = vb + nw * 2

                @pl.when(vb2 < n_blocks)
                def _():
                    gather(vb2, j)

            return carry

        lax.fori_loop(0, n_step, step, 0)
        for j in range(2):
            wait_stores(j)

        @pl.when(wid == 0)
        def _():
            pltpu.sync_copy(
                tail_hbm, buf_v.at[0, pl.ds(0, tail_v // 2), pl.ds(0, 128)]
            )
            pltpu.sync_copy(
                buf_v.at[0, pl.ds(0, tail_v // 2), pl.ds(0, 128)],
                out_hbm.at[pl.ds(n_blocks * (_VB // 2), tail_v // 2), :],
            )

    return relayout_kernel


@functools.lru_cache(maxsize=None)
def _build(n_pos: int, n_blocks: int):
    info = plsc.get_sparse_core_info()
    nw = info.num_cores * info.num_subcores
    assert n_blocks == nw and n_pos % 8 == 0
    n_pb = n_pos // 8
    mesh = plsc.VectorSubcoreMesh(core_axis_name="c", subcore_axis_name="s")

    @functools.partial(
        pl.kernel,
        mesh=mesh,
        out_type=jax.ShapeDtypeStruct(
            (n_pos, D_MODEL // 8, nw, 8, _CHUNK), jnp.float32
        ),
        scratch_types=[
            pltpu.VMEM((n_pb, 8, _CHUNK), jnp.int32),
            pltpu.VMEM((_NR, _CHUNK, D_MODEL), jnp.float32),
            pltpu.VMEM((_NO, D_MODEL // 8, 8, _PITCH), jnp.float32),
        ]
        + [pltpu.SemaphoreType.DMA] * (_NR + _NO),
        compiler_params=pltpu.CompilerParams(
            use_tc_tiling_on_sc=False, needs_layout_passes=False
        ),
    )
    def emb_kernel(xv_hbm, table_hbm, out_hbm, idx_v, rows_v, tile_v, *sems):
        rsems, osems = sems[:_NR], sems[_NR:]
        wid = lax.axis_index("s") * info.num_cores + lax.axis_index("c")
        pltpu.sync_copy(xv_hbm.at[:, wid], idx_v)

        iota = lax.iota(jnp.int32, _LANES)
        fbsel = [
            lax.shift_right_logical(g * _LANES + iota, 3)
            for g in range(D_MODEL // _LANES)
        ]
        fisel = [
            lax.bitwise_and(g * _LANES + iota, 7)
            for g in range(D_MODEL // _LANES)
        ]

        def gather(pb, pi, b):
            pltpu.async_copy(
                table_hbm.at[idx_v.at[pb, pi]], rows_v.at[b], rsems[b]
            )

        def wait_rows(b):
            pltpu.make_async_copy(
                table_hbm.at[pl.ds(0, _CHUNK)], rows_v.at[b], rsems[b]
            ).wait()

        def tile_out_src(o):
            return tile_v.at[o, :, :, pl.ds(0, _CHUNK)]

        def store(c, o):
            pltpu.async_copy(tile_out_src(o), out_hbm.at[c, :, wid], osems[o])

        def wait_store(o):
            pltpu.make_async_copy(
                tile_out_src(o), out_hbm.at[0, :, wid], osems[o]
            ).wait()

        def transpose_scale(b, o):
            @plsc.parallel_loop(0, _CHUNK, unroll=8)
            def row_body(r):
                bsel = jnp.full((_LANES,), r, jnp.int32)
                for g in range(D_MODEL // _LANES):
                    vals = rows_v[b, r, pl.ds(g * _LANES, _LANES)] * _SCALE
                    plsc.store_scatter(
                        tile_v.at[o], [fbsel[g], fisel[g], bsel], vals
                    )

        for j in range(_K):
            gather(0, j, j)

        def step(s, carry):
            for j in range(8):
                c = s * 8 + j
                b = j
                o = j % _NO
                wait_rows(b)

                @pl.when(c >= _NO)
                def _():
                    wait_store(o)

                transpose_scale(b, o)
                store(c, o)

                @pl.when(c + _K < n_pos)
                def _():
                    if j < 8 - _K:
                        gather(s, j + _K, (j + _K) % _NR)
                    else:
                        gather(s + 1, j + _K - 8, (j + _K) % _NR)

            return carry

        lax.fori_loop(0, n_pb, step, 0)
        for o in range(_NO):
            wait_store(o)

    return emb_kernel


def kernel(x, table):
    s0, s1 = x.shape
    info = plsc.get_sparse_core_info()
    nw = info.num_cores * info.num_subcores
    xv = (
        x.astype(jnp.int32)
        .reshape(nw, _CHUNK, s1 // 8, 8)
        .transpose(2, 0, 3, 1)
    )
    tail = table[_VOCAB - 64 :].reshape(32, 128)
    table_lin = _build_relayout()(table.T, tail).reshape(table.shape)
    out5 = _build(s1, nw)(xv, table_lin)
    out = out5.transpose(2, 4, 0, 1, 3).reshape(s0, s1, D_MODEL)
    return out


# speedup vs baseline: 1.3626x; 1.3626x over previous
import functools

import jax
import jax.numpy as jnp
from jax import lax
from jax.experimental import pallas as pl
from jax.experimental.pallas import tpu as pltpu
from jax.experimental.pallas import tpu_sc as plsc

D_MODEL = 64
_SCALE = 8.0
_CHUNK = 128
_LANES = 16
_NR = 8
_NO = 4
_K = 4
_PITCH = 129


@functools.lru_cache(maxsize=None)
def _build(n_pos: int, n_blocks: int):
    info = plsc.get_sparse_core_info()
    nw = info.num_cores * info.num_subcores
    assert n_blocks == nw and n_pos % 8 == 0
    n_pb = n_pos // 8
    mesh = plsc.VectorSubcoreMesh(core_axis_name="c", subcore_axis_name="s")

    @functools.partial(
        pl.kernel,
        mesh=mesh,
        out_type=jax.ShapeDtypeStruct(
            (n_pos, D_MODEL // 8, nw, 8, _CHUNK), jnp.float32
        ),
        scratch_types=[
            pltpu.VMEM((n_pb, 8, _CHUNK), jnp.int32),
            pltpu.VMEM((_NR, _CHUNK, D_MODEL), jnp.float32),
            pltpu.VMEM((_NO, D_MODEL // 8, 8, _PITCH), jnp.float32),
        ]
        + [pltpu.SemaphoreType.DMA] * (_NR + _NO),
        compiler_params=pltpu.CompilerParams(
            use_tc_tiling_on_sc=False, needs_layout_passes=False
        ),
    )
    def emb_kernel(xv_hbm, table_hbm, out_hbm, idx_v, rows_v, tile_v, *sems):
        rsems, osems = sems[:_NR], sems[_NR:]
        wid = lax.axis_index("s") * info.num_cores + lax.axis_index("c")
        pltpu.sync_copy(xv_hbm.at[:, wid], idx_v)

        iota = lax.iota(jnp.int32, _LANES)
        fbsel = [
            lax.shift_right_logical(g * _LANES + iota, 3)
            for g in range(D_MODEL // _LANES)
        ]
        fisel = [
            lax.bitwise_and(g * _LANES + iota, 7)
            for g in range(D_MODEL // _LANES)
        ]

        def gather(pb, pi, b):
            pltpu.async_copy(
                table_hbm.at[idx_v.at[pb, pi]], rows_v.at[b], rsems[b]
            )

        def wait_rows(b):
            pltpu.make_async_copy(
                table_hbm.at[pl.ds(0, _CHUNK)], rows_v.at[b], rsems[b]
            ).wait()

        def tile_out_src(o):
            return tile_v.at[o, :, :, pl.ds(0, _CHUNK)]

        def store(c, o):
            pltpu.async_copy(tile_out_src(o), out_hbm.at[c, :, wid], osems[o])

        def wait_store(o):
            pltpu.make_async_copy(
                tile_out_src(o), out_hbm.at[0, :, wid], osems[o]
            ).wait()

        def transpose_scale(b, o):
            @plsc.parallel_loop(0, _CHUNK, unroll=8)
            def row_body(r):
                bsel = jnp.full((_LANES,), r, jnp.int32)
                for g in range(D_MODEL // _LANES):
                    vals = rows_v[b, r, pl.ds(g * _LANES, _LANES)] * _SCALE
                    plsc.store_scatter(
                        tile_v.at[o], [fbsel[g], fisel[g], bsel], vals
                    )

        for j in range(_K):
            gather(0, j, j)

        def step(s, carry):
            for j in range(8):
                c = s * 8 + j
                b = j
                o = j % _NO
                wait_rows(b)

                @pl.when(c >= _NO)
                def _():
                    wait_store(o)

                transpose_scale(b, o)
                store(c, o)

                @pl.when(c + _K < n_pos)
                def _():
                    if j < 8 - _K:
                        gather(s, j + _K, (j + _K) % _NR)
                    else:
                        gather(s + 1, j + _K - 8, (j + _K) % _NR)

            return carry

        lax.fori_loop(0, n_pb, step, 0)
        for o in range(_NO):
            wait_store(o)

    return emb_kernel


def kernel(x, table):
    s0, s1 = x.shape
    info = plsc.get_sparse_core_info()
    nw = info.num_cores * info.num_subcores
    xv = (
        x.astype(jnp.int32)
        .reshape(nw, _CHUNK, s1 // 8, 8)
        .transpose(2, 0, 3, 1)
    )
    out5 = _build(s1, nw)(xv, table)
    out = out5.transpose(2, 4, 0, 1, 3).reshape(s0, s1, D_MODEL)
    return out
